# baseline (speedup 1.0000x reference)
import functools

import jax
import jax.numpy as jnp
from jax import lax
from jax.experimental import pallas as pl
from jax.experimental.pallas import tpu as pltpu
from jax.experimental.pallas import tpu_sc as plsc

N_NODES = 10000
F_IN = 256
E_EDGES = 160000
NCLS = 64

NC, NS = 2, 16
NW = NC * NS
CHUNK = 128
CHUNKS_PER_W = 40
E_PAD = NW * CHUNKS_PER_W * CHUNK
CH0 = 80
CH1 = 80 - CH0
MAXCH = max(CH0, CH1)
NJ = 10112
ROWS_PER_SUB = NJ // NS
BN = 2000
FW = 64
AW = 16


def _dot(a, b):
    return lax.dot_general(a, b, (((1,), (0,)), ((), ())),
                           precision=lax.Precision.HIGHEST,
                           preferred_element_type=jnp.float32)


def _tc1_body(x_ref, w_ref, ws_ref, wd_ref, h_ref, asrc_ref, adst_ref):
    x = x_ref[...]
    h_ref[...] = _dot(x, w_ref[...])
    asrc_ref[...] = _dot(x, ws_ref[...])
    adst_ref[...] = _dot(x, wd_ref[...])


def _tc2_body(m_ref, d_ref, r1_ref, w2_ref, ws2_ref, wd2_ref, b1_ref,
              h2_ref, asrc2_ref, adst2_ref):
    num = m_ref[0] + m_ref[1]
    den = d_ref[0] + d_ref[1]
    dinv = 1.0 / (den + 1e-16)
    o1 = num * _dot(dinv, r1_ref[...]) + b1_ref[...]
    e1 = jnp.where(o1 > 0, o1, jnp.exp(o1) - 1.0)
    h2_ref[...] = _dot(e1, w2_ref[...])
    asrc2_ref[...] = _dot(e1, ws2_ref[...])
    adst2_ref[...] = _dot(e1, wd2_ref[...])


def _tc3_body(m_ref, d_ref, c2_ref, b2_ref, out_ref):
    num = m_ref[0] + m_ref[1]
    den = d_ref[0] + d_ref[1]
    denb = _dot(den, c2_ref[...])
    o = num / (denb + 1e-16) + b2_ref[...]
    mx = jnp.max(o, axis=1, keepdims=True)
    s = o - mx
    lse = jnp.log(jnp.sum(jnp.exp(s), axis=1, keepdims=True))
    out_ref[...] = s - lse


def _edge_body(mode, hfeat, asrcp, adstp, srcs, dsts, outm, outd,
               accm, accd, idx_s, idx_d, rows, asr, adr, mrow, exr,
               gsem, ssem):
    c = lax.axis_index("c")
    s = lax.axis_index("s")
    wid = c * NS + s
    zb = s * ROWS_PER_SUB

    zero16 = jnp.zeros((16,), jnp.float32)

    def zrow(i, carry):
        for v in range(FW // 16):
            mrow[0][i, pl.ds(v * 16, 16)] = zero16
        exr[0][i, :] = zero16
        return carry

    lax.fori_loop(0, CHUNK, zrow, 0)

    for k in range(4):
        pltpu.sync_copy(mrow[0], accm.at[pl.ds(zb + k * CHUNK, CHUNK)])
        pltpu.sync_copy(exr[0], accd.at[pl.ds(zb + k * CHUNK, CHUNK)])
    rem = ROWS_PER_SUB - 4 * CHUNK
    pltpu.sync_copy(mrow[0].at[pl.ds(0, rem)],
                    accm.at[pl.ds(zb + 4 * CHUNK, rem)])
    pltpu.sync_copy(exr[0].at[pl.ds(0, rem)],
                    accd.at[pl.ds(zb + 4 * CHUNK, rem)])
    plsc.subcore_barrier()

    lane = lax.iota(jnp.int32, 16)
    if mode == "heads":
        exp_consts = [2 * v + jnp.right_shift(lane, 3) for v in range(4)]
    else:
        exp_consts = [lane * 0 for _ in range(4)]

    def issue_gather(j, b):
        pltpu.async_copy(hfeat.at[idx_s.at[j]], rows[b], gsem[b])
        pltpu.async_copy(asrcp.at[idx_s.at[j]], asr[b], gsem[b])
        pltpu.async_copy(adstp.at[idx_d.at[j]], adr[b], gsem[b])

    def wait_gather(b):
        pltpu.make_async_copy(hfeat.at[pl.ds(0, CHUNK)], rows[b], gsem[b]).wait()
        pltpu.make_async_copy(asrcp.at[pl.ds(0, CHUNK)], asr[b], gsem[b]).wait()
        pltpu.make_async_copy(adstp.at[pl.ds(0, CHUNK)], adr[b], gsem[b]).wait()

    def issue_scatter(j, b):
        pltpu.async_copy(mrow[b], accm.at[idx_d.at[j]], ssem[b], add=True)
        pltpu.async_copy(exr[b], accd.at[idx_d.at[j]], ssem[b], add=True)

    def wait_scatter(b):
        pltpu.make_async_copy(hfeat.at[pl.ds(0, CHUNK)], mrow[b], ssem[b]).wait()
        pltpu.make_async_copy(asrcp.at[pl.ds(0, CHUNK)], exr[b], ssem[b]).wait()

    def compute(b):
        rows_b, asr_b, adr_b, mrow_b, exr_b = (
            rows[b], asr[b], adr[b], mrow[b], exr[b])

        @plsc.parallel_loop(0, CHUNK, 1, unroll=4)
        def _(i):
            t = asr_b[i, :] + adr_b[i, :]
            al = jnp.where(t >= 0, t, 0.2 * t)
            ex = jnp.exp(al)
            for v in range(4):
                hv = rows_b[i, pl.ds(v * 16, 16)]
                pv = lax.gather(
                    ex, exp_consts[v][:, None],
                    lax.GatherDimensionNumbers(offset_dims=(),
                                               collapsed_slice_dims=(0,),
                                               start_index_map=(0,)),
                    slice_sizes=(1,),
                    mode=lax.GatherScatterMode.PROMISE_IN_BOUNDS)
                mrow_b[i, pl.ds(v * 16, 16)] = hv * pv
            exr_b[i, :] = ex

    def pipe(nch, g0):
        pltpu.sync_copy(srcs.at[pl.ds(g0, nch)], idx_s.at[pl.ds(0, nch)])
        pltpu.sync_copy(dsts.at[pl.ds(g0, nch)], idx_d.at[pl.ds(0, nch)])
        issue_gather(0, 0)
        issue_gather(1, 1)

        def outer(jj, carry):
            for b in range(2):
                j = 2 * jj + b
                wait_gather(b)

                @pl.when(jj > 0)
                def _():
                    wait_scatter(b)

                compute(b)
                issue_scatter(j, b)

                @pl.when(jj < nch // 2 - 1)
                def _():
                    issue_gather(j + 2, b)

            return carry

        lax.fori_loop(0, nch // 2, outer, 0)
        wait_scatter(0)
        wait_scatter(1)

    if CH0 > 0:
        @pl.when(c == 0)
        def _():
            pipe(CH0, s * CH0)

    if CH1 > 0:
        @pl.when(c == 1)
        def _():
            pipe(CH1, NS * CH0 + s * CH1)

    plsc.subcore_barrier()
    pltpu.sync_copy(accm.at[pl.ds(zb, ROWS_PER_SUB)],
                    outm.at[c, pl.ds(zb, ROWS_PER_SUB)])
    pltpu.sync_copy(accd.at[pl.ds(zb, ROWS_PER_SUB)],
                    outd.at[c, pl.ds(zb, ROWS_PER_SUB)])


def _make_edge_pass(mode):
    mesh = plsc.VectorSubcoreMesh(core_axis_name="c", subcore_axis_name="s",
                                  num_cores=NC, num_subcores=NS)
    return pl.kernel(
        functools.partial(_edge_body, mode),
        out_type=[
            jax.ShapeDtypeStruct((NC, NJ, FW), jnp.float32),
            jax.ShapeDtypeStruct((NC, NJ, AW), jnp.float32),
        ],
        mesh=mesh,
        scratch_types=[
            pltpu.VMEM_SHARED((NJ, FW), jnp.float32),
            pltpu.VMEM_SHARED((NJ, AW), jnp.float32),
            pltpu.VMEM((MAXCH, CHUNK), jnp.int32),
            pltpu.VMEM((MAXCH, CHUNK), jnp.int32),
            [pltpu.VMEM((CHUNK, FW), jnp.float32) for _ in range(2)],
            [pltpu.VMEM((CHUNK, AW), jnp.float32) for _ in range(2)],
            [pltpu.VMEM((CHUNK, AW), jnp.float32) for _ in range(2)],
            [pltpu.VMEM((CHUNK, FW), jnp.float32) for _ in range(2)],
            [pltpu.VMEM((CHUNK, AW), jnp.float32) for _ in range(2)],
            [pltpu.SemaphoreType.DMA for _ in range(2)],
            [pltpu.SemaphoreType.DMA for _ in range(2)],
        ],
        compiler_params=pltpu.CompilerParams(use_tc_tiling_on_sc=False),
        name="gat_edge_pass",
    )


def kernel(x, edge_index, W1, att_src1, att_dst1, b1,
           W2, att_src2, att_dst2, b2):
    f32 = jnp.float32
    H, C = att_src1.shape

    eye = jnp.eye(H, dtype=f32)
    As1 = (att_src1[:, :, None] * eye[:, None, :]).reshape(H * C, H)
    Ad1 = (att_dst1[:, :, None] * eye[:, None, :]).reshape(H * C, H)
    z8 = jnp.zeros((F_IN, 8), f32)
    Ws1p = jnp.concatenate([W1 @ As1, z8], axis=1)
    Wd1p = jnp.concatenate([W1 @ Ad1, z8], axis=1)

    As2 = att_src2.reshape(NCLS, 1)
    Ad2 = att_dst2.reshape(NCLS, 1)
    z15 = jnp.zeros((NCLS, 15), f32)
    Ws2p = jnp.concatenate([W2 @ As2, z15], axis=1)
    Wd2p = jnp.concatenate([W2 @ Ad2, z15], axis=1)

    R1 = jnp.concatenate([jnp.kron(eye, jnp.ones((1, C), f32)),
                          jnp.zeros((8, H * C), f32)], axis=0)
    C2 = jnp.zeros((16, NCLS), f32).at[0, :].set(1.0)

    b1r = b1.reshape(1, -1)
    b2r = b2.reshape(1, -1)

    pad = E_PAD - E_EDGES
    srcs = jnp.concatenate([edge_index[0], jnp.zeros((pad,), jnp.int32)])
    dsts = jnp.concatenate([edge_index[1],
                            jnp.full((pad,), N_NODES, jnp.int32)])
    srcs = srcs.reshape(NW * CHUNKS_PER_W, CHUNK)
    dsts = dsts.reshape(NW * CHUNKS_PER_W, CHUNK)

    grid = (N_NODES // BN,)

    def bs(shape):
        return pl.BlockSpec(shape, lambda i: tuple(0 for _ in shape))

    h1, asrc1p, adst1p = pl.pallas_call(
        _tc1_body,
        grid=grid,
        in_specs=[
            pl.BlockSpec((BN, F_IN), lambda i: (i, 0)),
            bs((F_IN, FW)), bs((F_IN, AW)), bs((F_IN, AW)),
        ],
        out_specs=[
            pl.BlockSpec((BN, FW), lambda i: (i, 0)),
            pl.BlockSpec((BN, AW), lambda i: (i, 0)),
            pl.BlockSpec((BN, AW), lambda i: (i, 0)),
        ],
        out_shape=[
            jax.ShapeDtypeStruct((N_NODES, FW), f32),
            jax.ShapeDtypeStruct((N_NODES, AW), f32),
            jax.ShapeDtypeStruct((N_NODES, AW), f32),
        ],
    )(x, W1, Ws1p, Wd1p)

    m1, d1 = _make_edge_pass("heads")(h1, asrc1p, adst1p, srcs, dsts)

    h2, asrc2p, adst2p = pl.pallas_call(
        _tc2_body,
        grid=grid,
        in_specs=[
            pl.BlockSpec((NC, BN, FW), lambda i: (0, i, 0)),
            pl.BlockSpec((NC, BN, AW), lambda i: (0, i, 0)),
            bs((16, 64)), bs((NCLS, FW)), bs((NCLS, AW)), bs((NCLS, AW)),
            bs((1, 64)),
        ],
        out_specs=[
            pl.BlockSpec((BN, FW), lambda i: (i, 0)),
            pl.BlockSpec((BN, AW), lambda i: (i, 0)),
            pl.BlockSpec((BN, AW), lambda i: (i, 0)),
        ],
        out_shape=[
            jax.ShapeDtypeStruct((N_NODES, FW), f32),
            jax.ShapeDtypeStruct((N_NODES, AW), f32),
            jax.ShapeDtypeStruct((N_NODES, AW), f32),
        ],
    )(m1, d1, R1, W2, Ws2p, Wd2p, b1r)

    m2, d2 = _make_edge_pass("bcast0")(h2, asrc2p, adst2p, srcs, dsts)

    out = pl.pallas_call(
        _tc3_body,
        grid=grid,
        in_specs=[
            pl.BlockSpec((NC, BN, FW), lambda i: (0, i, 0)),
            pl.BlockSpec((NC, BN, AW), lambda i: (0, i, 0)),
            bs((16, NCLS)), bs((1, NCLS)),
        ],
        out_specs=pl.BlockSpec((BN, NCLS), lambda i: (i, 0)),
        out_shape=jax.ShapeDtypeStruct((N_NODES, NCLS), f32),
    )(m2, d2, C2, b2r)

    return out


# speedup vs baseline: 1.2500x; 1.2500x over previous
import functools

import jax
import jax.numpy as jnp
from jax import lax
from jax.experimental import pallas as pl
from jax.experimental.pallas import tpu as pltpu
from jax.experimental.pallas import tpu_sc as plsc

N_NODES = 10000
F_IN = 256
E_EDGES = 160000
NCLS = 64

NC, NS = 2, 16
NW = NC * NS
CHUNK = 128
CHUNKS_PER_W = 40
E_PAD = NW * CHUNKS_PER_W * CHUNK
CH0 = 64
CH1 = 80 - CH0
MAXCH = max(CH0, CH1)
NJ = 10112
ROWS_PER_SUB = NJ // NS
BN = 2000
FW = 64
AW = 16


def _dot(a, b):
    return lax.dot_general(a, b, (((1,), (0,)), ((), ())),
                           precision=lax.Precision.HIGHEST,
                           preferred_element_type=jnp.float32)


def _tc1_body(x_ref, w_ref, ws_ref, wd_ref, h_ref, asrc_ref, adst_ref):
    x = x_ref[...]
    h_ref[...] = _dot(x, w_ref[...])
    asrc_ref[...] = _dot(x, ws_ref[...])
    adst_ref[...] = _dot(x, wd_ref[...])


def _tc2_body(m_ref, d_ref, r1_ref, w2_ref, ws2_ref, wd2_ref, b1_ref,
              h2_ref, asrc2_ref, adst2_ref):
    num = m_ref[0] + m_ref[1]
    den = d_ref[0] + d_ref[1]
    dinv = 1.0 / (den + 1e-16)
    o1 = num * _dot(dinv, r1_ref[...]) + b1_ref[...]
    e1 = jnp.where(o1 > 0, o1, jnp.exp(o1) - 1.0)
    h2_ref[...] = _dot(e1, w2_ref[...])
    asrc2_ref[...] = _dot(e1, ws2_ref[...])
    adst2_ref[...] = _dot(e1, wd2_ref[...])


def _tc3_body(m_ref, d_ref, c2_ref, b2_ref, out_ref):
    num = m_ref[0] + m_ref[1]
    den = d_ref[0] + d_ref[1]
    denb = _dot(den, c2_ref[...])
    o = num / (denb + 1e-16) + b2_ref[...]
    mx = jnp.max(o, axis=1, keepdims=True)
    s = o - mx
    lse = jnp.log(jnp.sum(jnp.exp(s), axis=1, keepdims=True))
    out_ref[...] = s - lse


def _edge_body(mode, hfeat, asrcp, adstp, srcs, dsts, outm, outd,
               accm, accd, idx_s, idx_d, rows, asr, adr, mrow, exr,
               gsem, ssem):
    c = lax.axis_index("c")
    s = lax.axis_index("s")
    wid = c * NS + s
    zb = s * ROWS_PER_SUB

    zero16 = jnp.zeros((16,), jnp.float32)

    def zrow(i, carry):
        for v in range(FW // 16):
            mrow[0][i, pl.ds(v * 16, 16)] = zero16
        exr[0][i, :] = zero16
        return carry

    lax.fori_loop(0, CHUNK, zrow, 0)

    for k in range(4):
        pltpu.sync_copy(mrow[0], accm.at[pl.ds(zb + k * CHUNK, CHUNK)])
        pltpu.sync_copy(exr[0], accd.at[pl.ds(zb + k * CHUNK, CHUNK)])
    rem = ROWS_PER_SUB - 4 * CHUNK
    pltpu.sync_copy(mrow[0].at[pl.ds(0, rem)],
                    accm.at[pl.ds(zb + 4 * CHUNK, rem)])
    pltpu.sync_copy(exr[0].at[pl.ds(0, rem)],
                    accd.at[pl.ds(zb + 4 * CHUNK, rem)])
    plsc.subcore_barrier()

    lane = lax.iota(jnp.int32, 16)
    if mode == "heads":
        exp_consts = [2 * v + jnp.right_shift(lane, 3) for v in range(4)]
    else:
        exp_consts = [lane * 0 for _ in range(4)]

    def issue_gather(j, b):
        pltpu.async_copy(hfeat.at[idx_s.at[j]], rows[b], gsem[b])
        pltpu.async_copy(asrcp.at[idx_s.at[j]], asr[b], gsem[b])
        pltpu.async_copy(adstp.at[idx_d.at[j]], adr[b], gsem[b])

    def wait_gather(b):
        pltpu.make_async_copy(hfeat.at[pl.ds(0, CHUNK)], rows[b], gsem[b]).wait()
        pltpu.make_async_copy(asrcp.at[pl.ds(0, CHUNK)], asr[b], gsem[b]).wait()
        pltpu.make_async_copy(adstp.at[pl.ds(0, CHUNK)], adr[b], gsem[b]).wait()

    def issue_scatter(j, b):
        pltpu.async_copy(mrow[b], accm.at[idx_d.at[j]], ssem[b], add=True)
        pltpu.async_copy(exr[b], accd.at[idx_d.at[j]], ssem[b], add=True)

    def wait_scatter(b):
        pltpu.make_async_copy(hfeat.at[pl.ds(0, CHUNK)], mrow[b], ssem[b]).wait()
        pltpu.make_async_copy(asrcp.at[pl.ds(0, CHUNK)], exr[b], ssem[b]).wait()

    def compute(b):
        rows_b, asr_b, adr_b, mrow_b, exr_b = (
            rows[b], asr[b], adr[b], mrow[b], exr[b])

        @plsc.parallel_loop(0, CHUNK, 1, unroll=4)
        def _(i):
            t = asr_b[i, :] + adr_b[i, :]
            al = jnp.where(t >= 0, t, 0.2 * t)
            ex = jnp.exp(al)
            for v in range(4):
                hv = rows_b[i, pl.ds(v * 16, 16)]
                pv = lax.gather(
                    ex, exp_consts[v][:, None],
                    lax.GatherDimensionNumbers(offset_dims=(),
                                               collapsed_slice_dims=(0,),
                                               start_index_map=(0,)),
                    slice_sizes=(1,),
                    mode=lax.GatherScatterMode.PROMISE_IN_BOUNDS)
                mrow_b[i, pl.ds(v * 16, 16)] = hv * pv
            exr_b[i, :] = ex

    def pipe(nch, g0):
        pltpu.sync_copy(srcs.at[pl.ds(g0, nch)], idx_s.at[pl.ds(0, nch)])
        pltpu.sync_copy(dsts.at[pl.ds(g0, nch)], idx_d.at[pl.ds(0, nch)])
        issue_gather(0, 0)
        issue_gather(1, 1)

        def outer(jj, carry):
            for b in range(2):
                j = 2 * jj + b
                wait_gather(b)

                @pl.when(jj > 0)
                def _():
                    wait_scatter(b)

                compute(b)
                issue_scatter(j, b)

                @pl.when(jj < nch // 2 - 1)
                def _():
                    issue_gather(j + 2, b)

            return carry

        lax.fori_loop(0, nch // 2, outer, 0)
        wait_scatter(0)
        wait_scatter(1)

    if CH0 > 0:
        @pl.when(c == 0)
        def _():
            pipe(CH0, s * CH0)

    if CH1 > 0:
        @pl.when(c == 1)
        def _():
            pipe(CH1, NS * CH0 + s * CH1)

    plsc.subcore_barrier()
    pltpu.sync_copy(accm.at[pl.ds(zb, ROWS_PER_SUB)],
                    outm.at[c, pl.ds(zb, ROWS_PER_SUB)])
    pltpu.sync_copy(accd.at[pl.ds(zb, ROWS_PER_SUB)],
                    outd.at[c, pl.ds(zb, ROWS_PER_SUB)])


def _make_edge_pass(mode):
    mesh = plsc.VectorSubcoreMesh(core_axis_name="c", subcore_axis_name="s",
                                  num_cores=NC, num_subcores=NS)
    return pl.kernel(
        functools.partial(_edge_body, mode),
        out_type=[
            jax.ShapeDtypeStruct((NC, NJ, FW), jnp.float32),
            jax.ShapeDtypeStruct((NC, NJ, AW), jnp.float32),
        ],
        mesh=mesh,
        scratch_types=[
            pltpu.VMEM_SHARED((NJ, FW), jnp.float32),
            pltpu.VMEM_SHARED((NJ, AW), jnp.float32),
            pltpu.VMEM((MAXCH, CHUNK), jnp.int32),
            pltpu.VMEM((MAXCH, CHUNK), jnp.int32),
            [pltpu.VMEM((CHUNK, FW), jnp.float32) for _ in range(2)],
            [pltpu.VMEM((CHUNK, AW), jnp.float32) for _ in range(2)],
            [pltpu.VMEM((CHUNK, AW), jnp.float32) for _ in range(2)],
            [pltpu.VMEM((CHUNK, FW), jnp.float32) for _ in range(2)],
            [pltpu.VMEM((CHUNK, AW), jnp.float32) for _ in range(2)],
            [pltpu.SemaphoreType.DMA for _ in range(2)],
            [pltpu.SemaphoreType.DMA for _ in range(2)],
        ],
        compiler_params=pltpu.CompilerParams(use_tc_tiling_on_sc=False),
        name="gat_edge_pass",
    )


def kernel(x, edge_index, W1, att_src1, att_dst1, b1,
           W2, att_src2, att_dst2, b2):
    f32 = jnp.float32
    H, C = att_src1.shape

    eye = jnp.eye(H, dtype=f32)
    As1 = (att_src1[:, :, None] * eye[:, None, :]).reshape(H * C, H)
    Ad1 = (att_dst1[:, :, None] * eye[:, None, :]).reshape(H * C, H)
    z8 = jnp.zeros((F_IN, 8), f32)
    Ws1p = jnp.concatenate([W1 @ As1, z8], axis=1)
    Wd1p = jnp.concatenate([W1 @ Ad1, z8], axis=1)

    As2 = att_src2.reshape(NCLS, 1)
    Ad2 = att_dst2.reshape(NCLS, 1)
    z15 = jnp.zeros((NCLS, 15), f32)
    Ws2p = jnp.concatenate([W2 @ As2, z15], axis=1)
    Wd2p = jnp.concatenate([W2 @ Ad2, z15], axis=1)

    R1 = jnp.concatenate([jnp.kron(eye, jnp.ones((1, C), f32)),
                          jnp.zeros((8, H * C), f32)], axis=0)
    C2 = jnp.zeros((16, NCLS), f32).at[0, :].set(1.0)

    b1r = b1.reshape(1, -1)
    b2r = b2.reshape(1, -1)

    pad = E_PAD - E_EDGES
    srcs = jnp.concatenate([edge_index[0], jnp.zeros((pad,), jnp.int32)])
    dsts = jnp.concatenate([edge_index[1],
                            jnp.full((pad,), N_NODES, jnp.int32)])
    srcs = srcs.reshape(NW * CHUNKS_PER_W, CHUNK)
    dsts = dsts.reshape(NW * CHUNKS_PER_W, CHUNK)

    grid = (N_NODES // BN,)

    def bs(shape):
        return pl.BlockSpec(shape, lambda i: tuple(0 for _ in shape))

    h1, asrc1p, adst1p = pl.pallas_call(
        _tc1_body,
        grid=grid,
        in_specs=[
            pl.BlockSpec((BN, F_IN), lambda i: (i, 0)),
            bs((F_IN, FW)), bs((F_IN, AW)), bs((F_IN, AW)),
        ],
        out_specs=[
            pl.BlockSpec((BN, FW), lambda i: (i, 0)),
            pl.BlockSpec((BN, AW), lambda i: (i, 0)),
            pl.BlockSpec((BN, AW), lambda i: (i, 0)),
        ],
        out_shape=[
            jax.ShapeDtypeStruct((N_NODES, FW), f32),
            jax.ShapeDtypeStruct((N_NODES, AW), f32),
            jax.ShapeDtypeStruct((N_NODES, AW), f32),
        ],
    )(x, W1, Ws1p, Wd1p)

    m1, d1 = _make_edge_pass("heads")(h1, asrc1p, adst1p, srcs, dsts)

    h2, asrc2p, adst2p = pl.pallas_call(
        _tc2_body,
        grid=grid,
        in_specs=[
            pl.BlockSpec((NC, BN, FW), lambda i: (0, i, 0)),
            pl.BlockSpec((NC, BN, AW), lambda i: (0, i, 0)),
            bs((16, 64)), bs((NCLS, FW)), bs((NCLS, AW)), bs((NCLS, AW)),
            bs((1, 64)),
        ],
        out_specs=[
            pl.BlockSpec((BN, FW), lambda i: (i, 0)),
            pl.BlockSpec((BN, AW), lambda i: (i, 0)),
            pl.BlockSpec((BN, AW), lambda i: (i, 0)),
        ],
        out_shape=[
            jax.ShapeDtypeStruct((N_NODES, FW), f32),
            jax.ShapeDtypeStruct((N_NODES, AW), f32),
            jax.ShapeDtypeStruct((N_NODES, AW), f32),
        ],
    )(m1, d1, R1, W2, Ws2p, Wd2p, b1r)

    m2, d2 = _make_edge_pass("bcast0")(h2, asrc2p, adst2p, srcs, dsts)

    out = pl.pallas_call(
        _tc3_body,
        grid=grid,
        in_specs=[
            pl.BlockSpec((NC, BN, FW), lambda i: (0, i, 0)),
            pl.BlockSpec((NC, BN, AW), lambda i: (0, i, 0)),
            bs((16, NCLS)), bs((1, NCLS)),
        ],
        out_specs=pl.BlockSpec((BN, NCLS), lambda i: (i, 0)),
        out_shape=jax.ShapeDtypeStruct((N_NODES, NCLS), f32),
    )(m2, d2, C2, b2r)

    return out
